# baseline (speedup 1.0000x reference)
import functools

import jax
import jax.numpy as jnp
import numpy as np
from jax.experimental import pallas as pl
from jax.experimental.pallas import tpu as pltpu

NODE_TYPES = ('inst', 'var', 'const', 'array')
D_FEAT = 128
N_OUT = 128
N_HID = 128
HEADS_ATT = 8
N_HID_SET = 128
HEADS_SET = 4
NIP = 16
NEG = 0.1


def _mm_kernel(x_ref, w_ref, b_ref, o_ref):
    o_ref[...] = (
        jnp.dot(x_ref[...], w_ref[...], preferred_element_type=jnp.float32)
        + b_ref[...]
    )


def matmul_bias(x, w_t, b, block_n=512):
    N, K = x.shape
    M = w_t.shape[1]
    pad = (-N) % block_n
    if pad:
        x = jnp.pad(x, ((0, pad), (0, 0)))
    Np = x.shape[0]
    out = pl.pallas_call(
        _mm_kernel,
        grid=(Np // block_n,),
        in_specs=[
            pl.BlockSpec((block_n, K), lambda i: (i, 0)),
            pl.BlockSpec((K, M), lambda i: (0, 0)),
            pl.BlockSpec((M,), lambda i: (0,)),
        ],
        out_specs=pl.BlockSpec((block_n, M), lambda i: (i, 0)),
        out_shape=jax.ShapeDtypeStruct((Np, M), jnp.float32),
    )(x, w_t, b)
    return out[:N] if pad else out


def _layer_norm(x, g, b):
    m = x.mean(-1, keepdims=True)
    v = ((x - m) ** 2).mean(-1, keepdims=True)
    return (x - m) / jnp.sqrt(v + 1e-5) * g + b


def _han_conv(p, x_dict, ei_dict, heads, out_ch):
    D = out_ch // heads
    xp = {}
    for t, x in x_dict.items():
        W, b = p['proj'][t]
        xp[t] = matmul_bias(x, W.T, b).reshape(-1, heads, D)
    outs = {t: [] for t in x_dict}
    for et, ei in ei_dict.items():
        s, _, d = et
        if s not in xp or d not in xp:
            continue
        a_s, a_d = p['att']['__'.join(et)]
        asrc = (xp[s] * a_s).sum(-1)
        adst = (xp[d] * a_d).sum(-1)
        src, dst = ei[0], ei[1]
        alpha = jax.nn.leaky_relu(asrc[src] + adst[dst], NEG)
        n_dst = xp[d].shape[0]
        amax = jax.ops.segment_max(alpha, dst, num_segments=n_dst)
        amax = jnp.where(jnp.isfinite(amax), amax, 0.0)
        ex = jnp.exp(alpha - amax[dst])
        den = jax.ops.segment_sum(ex, dst, num_segments=n_dst)
        w = ex / (den[dst] + 1e-16)
        msg = xp[s][src] * w[:, :, None]
        agg = jax.ops.segment_sum(msg, dst, num_segments=n_dst).reshape(n_dst, out_ch)
        outs[d].append(jax.nn.relu(agg))
    res = {}
    Wk, bk = p['k_lin']
    q = p['q']
    for t, lst in outs.items():
        if not lst:
            res[t] = None
            continue
        O = jnp.stack(lst)
        score = (q * jnp.tanh(O @ Wk.T + bk).mean(axis=1)).sum(-1)
        a = jax.nn.softmax(score, axis=0)
        res[t] = (a[:, None, None] * O).sum(0)
    return res


def _mha(q_in, kv, in_w, in_b, out_w, out_b, heads):
    d = q_in.shape[-1]
    Wq, Wk, Wv = in_w[:d], in_w[d:2 * d], in_w[2 * d:]
    bq, bk, bv = in_b[:d], in_b[d:2 * d], in_b[2 * d:]
    q = q_in @ Wq.T + bq
    k = matmul_bias(kv, Wk.T, bk)
    v = matmul_bias(kv, Wv.T, bv)
    L, S = q.shape[0], k.shape[0]
    hd = d // heads
    q = q.reshape(L, heads, hd).transpose(1, 0, 2)
    k = k.reshape(S, heads, hd).transpose(1, 0, 2)
    v = v.reshape(S, heads, hd).transpose(1, 0, 2)
    att = jax.nn.softmax(q @ k.transpose(0, 2, 1) / np.sqrt(hd).astype(np.float32), axis=-1)
    o = (att @ v).transpose(1, 0, 2).reshape(L, d)
    return o @ out_w.T + out_b


def _set_transformer(p, x):
    ow, ob = p['out']
    o = _mha(p['inducing'], x, p['in_w'], p['in_b'], ow, ob, HEADS_SET)
    fw, fb = p['fc']
    return o.mean(axis=0) @ fw.T + fb


def kernel(x_dict, edge_index_dict, params):
    ei = dict(edge_index_dict)
    h = _han_conv(params['han1'], x_dict, ei, HEADS_ATT, N_HID)
    h = {t: jax.nn.elu(h[t]) for t in ('inst', 'var', 'array')}
    ei.pop(('const', 'data', 'inst'), None)
    g, b = params['norm1']
    h = {t: _layer_norm(v, g, b) for t, v in h.items()}
    h = _han_conv(params['han2'], h, ei, HEADS_ATT, N_HID)
    h = {t: jax.nn.elu(h[t]) for t in ('inst', 'var', 'array')}
    g, b = params['norm2']
    h = {t: _layer_norm(v, g, b) for t, v in h.items()}
    h = _han_conv(params['han3'], h, ei, HEADS_SET, N_HID_SET)
    h = {t: jax.nn.elu(h[t]) for t in ('inst', 'var', 'array')}
    aggs = []
    for t, st, fc in (('inst', 'st_inst', 'fc_inst'), ('var', 'st_var', 'fc_var'),
                      ('array', 'st_array', 'fc_array')):
        w, bb = params[fc]
        aggs.append(jax.nn.gelu(_set_transformer(params[st], h[t]) @ w.T + bb,
                                approximate=False))
    cat = jnp.concatenate(aggs, axis=-1)
    wg, bg = params['fc_graph']
    return jax.nn.relu(cat) @ wg.T + bg


# speedup vs baseline: 17.5500x; 17.5500x over previous
import functools

import jax
import jax.numpy as jnp
import numpy as np
from jax import lax
from jax.experimental import pallas as pl
from jax.experimental.pallas import tpu as pltpu
from jax.experimental.pallas import tpu_sc as plsc

BN = 512
SUB = 128
NT = 16
NEG = 0.1
HEADS_SET = 4
F32 = jnp.float32


def _rup(a, b):
    return (a + b - 1) // b * b


def _proj_body(x_ref, wt_ref, b_ref, as_ref, ad_ref, *outs):
    xp = jnp.dot(x_ref[...], wt_ref[...], preferred_element_type=F32) + b_ref[...]
    for c in range(8):
        outs[c][...] = xp[:, 16 * c:16 * c + 16]
    outs[8][...] = jnp.dot(xp, as_ref[...], preferred_element_type=F32)
    outs[9][...] = jnp.dot(xp, ad_ref[...], preferred_element_type=F32)


def _proj_call(x, wt, b, As, Ad):
    NP = x.shape[0]
    Rs, Rd = As.shape[1], Ad.shape[1]
    outs = ([jax.ShapeDtypeStruct((NP, 16), F32)] * 8
            + [jax.ShapeDtypeStruct((NP, Rs), F32),
               jax.ShapeDtypeStruct((NP, Rd), F32)])
    out_specs = ([pl.BlockSpec((BN, 16), lambda i: (i, 0))] * 8
                 + [pl.BlockSpec((BN, Rs), lambda i: (i, 0)),
                    pl.BlockSpec((BN, Rd), lambda i: (i, 0))])
    return pl.pallas_call(
        _proj_body,
        grid=(NP // BN,),
        in_specs=[pl.BlockSpec((BN, 128), lambda i: (i, 0)),
                  pl.BlockSpec((128, 128), lambda i: (0, 0)),
                  pl.BlockSpec((1, 128), lambda i: (0, 0)),
                  pl.BlockSpec((128, Rs), lambda i: (0, 0)),
                  pl.BlockSpec((128, Rd), lambda i: (0, 0))],
        out_specs=out_specs,
        out_shape=outs,
    )(x, wt, b, As, Ad)


@functools.lru_cache(maxsize=None)
def _make_sc_den(Ep, NPs, NPd, H):
    nk = Ep // (NT * SUB)
    rows_pt = NPd // NT
    ZR = 1024
    mesh = plsc.VectorSubcoreMesh(core_axis_name="c", subcore_axis_name="s",
                                  num_cores=2, num_subcores=16)

    def body(src_hbm, dst_hbm, asrc_hbm, adst_hbm, den_out, ex_out,
             srcv, dstv, asb, adb, exb, zb, den_sh, sem1, sem2):
        cid = lax.axis_index("c")
        sid = lax.axis_index("s")
        zeros16 = jnp.zeros((16,), F32)

        @pl.loop(0, ZR)
        def _z(r):
            zb[r] = zeros16

        base = sid * rows_pt
        nfull, tail = rows_pt // ZR, rows_pt % ZR
        for q in range(nfull):
            pltpu.sync_copy(zb, den_sh.at[pl.ds(base + q * ZR, ZR)])
        if tail:
            pltpu.sync_copy(zb.at[pl.ds(0, tail)],
                            den_sh.at[pl.ds(base + nfull * ZR, tail)])
        plsc.subcore_barrier()

        pltpu.sync_copy(src_hbm.at[sid], srcv)
        pltpu.sync_copy(dst_hbm.at[sid], dstv)

        @pl.loop(0, nk)
        def _k(k):
            cp1 = pltpu.async_copy(asrc_hbm.at[srcv.at[k]], asb, sem1)
            cp2 = pltpu.async_copy(adst_hbm.at[dstv.at[k]], adb, sem2)
            cp1.wait()
            cp2.wait()

            @pl.loop(0, SUB, step=8)
            def _e(e0):
                for de in range(8):
                    e = e0 + de
                    a = asb[e] + adb[e]
                    a = jnp.where(a > 0, a, a * NEG)
                    exb[e] = jnp.exp(a)

            @pl.when(cid == 0)
            def _w():
                pltpu.sync_copy(exb, ex_out.at[pl.ds((sid * nk + k) * SUB, SUB)])

            pltpu.sync_copy(exb, den_sh.at[dstv.at[k]], add=True)

        plsc.subcore_barrier()

        @pl.when(cid == 0)
        def _dump():
            pltpu.sync_copy(den_sh.at[pl.ds(sid * rows_pt, rows_pt)],
                            den_out.at[pl.ds(sid * rows_pt, rows_pt)])

    return pl.kernel(
        body,
        out_type=[jax.ShapeDtypeStruct((NPd, 16), F32),
                  jax.ShapeDtypeStruct((Ep, 16), F32)],
        mesh=mesh,
        compiler_params=pltpu.CompilerParams(use_tc_tiling_on_sc=False),
        scratch_types=[pltpu.VMEM((nk, SUB), jnp.int32),
                       pltpu.VMEM((nk, SUB), jnp.int32),
                       pltpu.VMEM((SUB, 16), F32),
                       pltpu.VMEM((SUB, 16), F32),
                       pltpu.VMEM((SUB, 16), F32),
                       pltpu.VMEM((ZR, 16), F32),
                       pltpu.VMEM_SHARED((NPd, 16), F32),
                       pltpu.SemaphoreType.DMA,
                       pltpu.SemaphoreType.DMA],
    )


@functools.lru_cache(maxsize=None)
def _make_sc_agg(Ep, NPs, NPd, H):
    nk = Ep // (NT * SUB)
    rows_pt = NPd // NT
    hdiv = 8 // H
    ZR = 1024
    mesh = plsc.VectorSubcoreMesh(core_axis_name="c", subcore_axis_name="s",
                                  num_cores=2, num_subcores=16)

    def body(src_hbm, dst_hbm, ex_hbm, den_hbm,
             xp0, xp1, xp2, xp3, xp4, xp5, xp6, xp7,
             w0, w1, agg0, agg1, agg2, agg3, agg4, agg5, agg6, agg7,
             srcv, dstv, exb, dnb, wb, xpb, msgb, zb, agg_sh, sem1):
        cid = lax.axis_index("c")
        sid = lax.axis_index("s")
        zeros16 = jnp.zeros((16,), F32)

        pltpu.sync_copy(src_hbm.at[sid], srcv)
        pltpu.sync_copy(dst_hbm.at[sid], dstv)

        @pl.loop(0, ZR)
        def _z(r):
            zb[r] = zeros16

        def half(w_out, parts):
            @pl.loop(0, nk)
            def _kw(k):
                pltpu.async_copy(den_hbm.at[dstv.at[k]], dnb, sem1).wait()
                pltpu.sync_copy(ex_hbm.at[pl.ds((sid * nk + k) * SUB, SUB)], exb)

                @pl.loop(0, SUB, step=8)
                def _e(e0):
                    for de in range(8):
                        e = e0 + de
                        wb[e] = exb[e] / (dnb[e] + 1e-16)

                pltpu.sync_copy(wb, w_out.at[pl.ds((sid * nk + k) * SUB, SUB)])

            for xp_ref, agg_ref, cc in parts:
                head = cc // hdiv
                base = sid * rows_pt
                nfull, tail = rows_pt // ZR, rows_pt % ZR
                for q in range(nfull):
                    pltpu.sync_copy(zb, agg_sh.at[pl.ds(base + q * ZR, ZR)])
                if tail:
                    pltpu.sync_copy(zb.at[pl.ds(0, tail)],
                                    agg_sh.at[pl.ds(base + nfull * ZR, tail)])
                plsc.subcore_barrier()

                @pl.loop(0, nk)
                def _k(k):
                    cp = pltpu.async_copy(xp_ref.at[srcv.at[k]], xpb, sem1)
                    pltpu.sync_copy(
                        w_out.at[pl.ds((sid * nk + k) * SUB, SUB)], wb)
                    cp.wait()

                    @pl.loop(0, SUB, step=8)
                    def _e(e0):
                        for de in range(8):
                            e = e0 + de
                            ws = jnp.broadcast_to(wb[e][head], (16,))
                            msgb[e] = xpb[e] * ws

                    pltpu.sync_copy(msgb, agg_sh.at[dstv.at[k]], add=True)

                plsc.subcore_barrier()
                pltpu.sync_copy(agg_sh.at[pl.ds(sid * rows_pt, rows_pt)],
                                agg_ref.at[pl.ds(sid * rows_pt, rows_pt)])

        @pl.when(cid == 0)
        def _c0():
            half(w0, [(xp0, agg0, 0), (xp1, agg1, 1),
                      (xp2, agg2, 2), (xp3, agg3, 3)])

        @pl.when(cid == 1)
        def _c1():
            half(w1, [(xp4, agg4, 4), (xp5, agg5, 5),
                      (xp6, agg6, 6), (xp7, agg7, 7)])

    return pl.kernel(
        body,
        out_type=[jax.ShapeDtypeStruct((Ep, 16), F32)] * 2
                 + [jax.ShapeDtypeStruct((NPd, 16), F32) for _ in range(8)],
        mesh=mesh,
        compiler_params=pltpu.CompilerParams(use_tc_tiling_on_sc=False),
        scratch_types=[pltpu.VMEM((nk, SUB), jnp.int32),
                       pltpu.VMEM((nk, SUB), jnp.int32),
                       pltpu.VMEM((SUB, 16), F32),
                       pltpu.VMEM((SUB, 16), F32),
                       pltpu.VMEM((SUB, 16), F32),
                       pltpu.VMEM((SUB, 16), F32),
                       pltpu.VMEM((SUB, 16), F32),
                       pltpu.VMEM((ZR, 16), F32),
                       pltpu.VMEM_SHARED((NPd, 16), F32),
                       pltpu.SemaphoreType.DMA],
    )


def _sc_agg_call(Ep, NPs, NPd, H, src, dst, ex, den, xpc):
    outs = _make_sc_agg(Ep, NPs, NPd, H)(src, dst, ex, den, *xpc)
    return outs[2:]


def _make_sem_body(K, Nd):
    def body(*refs):
        wk_ref, bk_ref = refs[0], refs[1]
        chunks = refs[2:2 + 8 * K]
        out_ref = refs[2 + 8 * K]
        i = pl.program_id(0)

        @pl.when(i == 0)
        def _init():
            out_ref[...] = jnp.zeros_like(out_ref)

        rows = i * BN + lax.broadcasted_iota(jnp.int32, (BN, 1), 0)
        mask = rows < Nd
        for k in range(K):
            x = jnp.concatenate([chunks[8 * k + c][...] for c in range(8)], axis=1)
            t = jnp.tanh(jnp.dot(jnp.maximum(x, 0.0), wk_ref[...],
                                 preferred_element_type=F32) + bk_ref[...])
            t = jnp.where(mask, t, 0.0)
            out_ref[k:k + 1, :] += jnp.sum(t, axis=0, keepdims=True)
    return body


def _sem_call(agg_chunks, wkT, bk, Nd, NP):
    K = len(agg_chunks)
    flat = [c for oct_ in agg_chunks for c in oct_]
    in_specs = ([pl.BlockSpec((128, 128), lambda i: (0, 0)),
                 pl.BlockSpec((1, 128), lambda i: (0, 0))]
                + [pl.BlockSpec((BN, 16), lambda i: (i, 0))] * (8 * K))
    return pl.pallas_call(
        _make_sem_body(K, Nd),
        grid=(NP // BN,),
        in_specs=in_specs,
        out_specs=pl.BlockSpec((K, 128), lambda i: (0, 0)),
        out_shape=jax.ShapeDtypeStruct((K, 128), F32),
    )(wkT, bk, *flat)


def _make_combine_body(K, Nd, do_norm):
    def body(*refs):
        st_ref, q_ref, g_ref, bb_ref = refs[:4]
        chunks = refs[4:4 + 8 * K]
        h_ref = refs[4 + 8 * K]
        score = jnp.sum(st_ref[...] * q_ref[...], axis=1, keepdims=True) / Nd
        score = score - jnp.max(score, axis=0, keepdims=True)
        a = jnp.exp(score)
        a = a / jnp.sum(a, axis=0, keepdims=True)
        acc = jnp.zeros((BN, 128), F32)
        for k in range(K):
            x = jnp.concatenate([chunks[8 * k + c][...] for c in range(8)], axis=1)
            acc = acc + a[k:k + 1, 0:1] * jnp.maximum(x, 0.0)
        acc = jnp.where(acc > 0, acc, jnp.exp(jnp.minimum(acc, 0.0)) - 1.0)
        if do_norm:
            m = jnp.mean(acc, axis=1, keepdims=True)
            v = jnp.mean((acc - m) ** 2, axis=1, keepdims=True)
            acc = (acc - m) / jnp.sqrt(v + 1e-5) * g_ref[...] + bb_ref[...]
        h_ref[...] = acc
    return body


def _combine_call(agg_chunks, sum_tanh, q, g, bnorm, Nd, NP, do_norm):
    K = len(agg_chunks)
    flat = [c for oct_ in agg_chunks for c in oct_]
    in_specs = ([pl.BlockSpec((K, 128), lambda i: (0, 0)),
                 pl.BlockSpec((1, 128), lambda i: (0, 0)),
                 pl.BlockSpec((1, 128), lambda i: (0, 0)),
                 pl.BlockSpec((1, 128), lambda i: (0, 0))]
                + [pl.BlockSpec((BN, 16), lambda i: (i, 0))] * (8 * K))
    return pl.pallas_call(
        _make_combine_body(K, Nd, do_norm),
        grid=(NP // BN,),
        in_specs=in_specs,
        out_specs=pl.BlockSpec((BN, 128), lambda i: (i, 0)),
        out_shape=jax.ShapeDtypeStruct((NP, 128), F32),
    )(sum_tanh, q, g, bnorm, *flat)


def _make_st_body(N, nb):
    isq = np.float32(1.0 / np.sqrt(32.0))

    def body(h_ref, wq, bq, wk, bk, wv, bv, ind_ref, wo, bo,
             out_ref, m_s, l_s, o_s):
        i = pl.program_id(0)

        @pl.when(i == 0)
        def _init():
            m_s[...] = jnp.full((16, 8), -1e30, F32)
            l_s[...] = jnp.zeros((16, 8), F32)
            o_s[...] = jnp.zeros((16, 128), F32)

        q = jnp.dot(ind_ref[...], wq[...], preferred_element_type=F32) + bq[...]
        k = jnp.dot(h_ref[...], wk[...], preferred_element_type=F32) + bk[...]
        v = jnp.dot(h_ref[...], wv[...], preferred_element_type=F32) + bv[...]
        cols = i * BN + lax.broadcasted_iota(jnp.int32, (16, BN), 1)
        cmask = cols < N
        for hh in range(4):
            qh = q[:, 32 * hh:32 * hh + 32]
            kh = k[:, 32 * hh:32 * hh + 32]
            vh = v[:, 32 * hh:32 * hh + 32]
            s = lax.dot_general(qh, kh, (((1,), (1,)), ((), ())),
                                preferred_element_type=F32) * isq
            s = jnp.where(cmask, s, -1e30)
            mo = m_s[:, hh:hh + 1]
            lo = l_s[:, hh:hh + 1]
            oo = o_s[:, 32 * hh:32 * hh + 32]
            mn = jnp.maximum(mo, jnp.max(s, axis=1, keepdims=True))
            corr = jnp.exp(mo - mn)
            p = jnp.exp(s - mn)
            m_s[:, hh:hh + 1] = mn
            l_s[:, hh:hh + 1] = lo * corr + jnp.sum(p, axis=1, keepdims=True)
            o_s[:, 32 * hh:32 * hh + 32] = oo * corr + jnp.dot(
                p, vh, preferred_element_type=F32)

        @pl.when(i == nb - 1)
        def _fin():
            lfull = jnp.concatenate(
                [jnp.broadcast_to(l_s[:, hh:hh + 1], (16, 32)) for hh in range(4)],
                axis=1)
            out_ref[...] = jnp.dot(o_s[...] / lfull, wo[...],
                                   preferred_element_type=F32) + bo[...]

    return body


def _st_call(h, N, NP, wqT, bq, wkT, bk, wvT, bv, ind, woT, bo):
    nb = NP // BN
    mat = pl.BlockSpec((128, 128), lambda i: (0, 0))
    vec = pl.BlockSpec((1, 128), lambda i: (0, 0))
    return pl.pallas_call(
        _make_st_body(N, nb),
        grid=(nb,),
        in_specs=[pl.BlockSpec((BN, 128), lambda i: (i, 0)),
                  mat, vec, mat, vec, mat, vec,
                  pl.BlockSpec((16, 128), lambda i: (0, 0)),
                  mat, vec],
        out_specs=pl.BlockSpec((16, 128), lambda i: (0, 0)),
        out_shape=jax.ShapeDtypeStruct((16, 128), F32),
        scratch_shapes=[pltpu.VMEM((16, 8), F32),
                        pltpu.VMEM((16, 8), F32),
                        pltpu.VMEM((16, 128), F32)],
    )(h, wqT, bq, wkT, bk, wvT, bv, ind, woT, bo)


def _head_body(oi, ov, oa, fwi, fbi, cwi, cbi, fwv, fbv, cwv, cbv,
               fwa, fba, cwa, cba, wg, bg, out_ref):
    parts = []
    for o, fw, fb, cw, cb in ((oi, fwi, fbi, cwi, cbi),
                              (ov, fwv, fbv, cwv, cbv),
                              (oa, fwa, fba, cwa, cba)):
        s = jnp.mean(o[...], axis=0, keepdims=True)
        z = jnp.dot(s, fw[...], preferred_element_type=F32) + fb[...]
        z = jnp.dot(z, cw[...], preferred_element_type=F32) + cb[...]
        z = 0.5 * z * (1.0 + lax.erf(z * np.float32(1.0 / np.sqrt(2.0))))
        parts.append(z)
    cat = jnp.maximum(jnp.concatenate(parts, axis=1), 0.0)
    out_ref[...] = jnp.dot(cat, wg[...], preferred_element_type=F32) + bg[...]


def _head_call(os_, fcs, cws, wgT, bg):
    ins = []
    for o, (fw, fb), (cw, cb) in zip(os_, fcs, cws):
        ins.extend([fw, fb, cw, cb])
    return pl.pallas_call(
        _head_body,
        out_shape=jax.ShapeDtypeStruct((1, 128), F32),
    )(os_[0], os_[1], os_[2], *ins, wgT, bg)


def _att_mat(a, H, D):
    sel = np.repeat(np.arange(H), D)
    eye = np.zeros((128, 16), np.float32)
    eye[np.arange(128), sel] = 1.0
    return a.reshape(-1)[:, None] * jnp.asarray(eye)


def kernel(x_dict, edge_index_dict, params):
    types = list(x_dict.keys())
    N = {t: x_dict[t].shape[0] for t in types}
    NP = {t: _rup(N[t] + 1, BN) for t in types}

    eprep = {}
    for et, ei in edge_index_dict.items():
        s, _, d = et
        E = ei.shape[1]
        Ep = _rup(E, NT * SUB)
        nk = Ep // (NT * SUB)
        src = jnp.pad(ei[0], (0, Ep - E), constant_values=N[s]).reshape(NT, nk, SUB)
        dst = jnp.pad(ei[1], (0, Ep - E), constant_values=N[d]).reshape(NT, nk, SUB)
        eprep[et] = (src, dst, Ep)

    h = {t: jnp.pad(x_dict[t], ((0, NP[t] - N[t]), (0, 0))) for t in types}

    for li, (pname, H) in enumerate((('han1', 8), ('han2', 8), ('han3', HEADS_SET))):
        hp = params[pname]
        D = 128 // H
        cur = list(h.keys())
        ets = [et for et in edge_index_dict
               if et[0] in h and et[2] in h]
        src_roles = {t: [et for et in ets if et[0] == t] for t in cur}
        dst_roles = {t: [et for et in ets if et[2] == t] for t in cur}

        xp_chunks, asrc_tab, adst_tab = {}, {}, {}
        for t in cur:
            W, b = hp['proj'][t]
            As_list = [_att_mat(hp['att']['__'.join(et)][0], H, D)
                       for et in src_roles[t]]
            Ad_list = [_att_mat(hp['att']['__'.join(et)][1], H, D)
                       for et in dst_roles[t]]
            As = (jnp.concatenate(As_list, axis=1) if As_list
                  else jnp.zeros((128, 16), F32))
            Ad = (jnp.concatenate(Ad_list, axis=1) if Ad_list
                  else jnp.zeros((128, 16), F32))
            o = _proj_call(h[t], W.T, b.reshape(1, 128), As, Ad)
            xp_chunks[t] = o[0:8]
            asrc_all, adst_all = o[8], o[9]
            for i, et in enumerate(src_roles[t]):
                asrc_tab[et] = asrc_all[:, i * 16:(i + 1) * 16]
            for i, et in enumerate(dst_roles[t]):
                adst_tab[et] = adst_all[:, i * 16:(i + 1) * 16]

        agg = {}
        for et in ets:
            s, _, d = et
            src, dst, Ep = eprep[et]
            den, ex = _make_sc_den(Ep, NP[s], NP[d], H)(
                src, dst, asrc_tab[et], adst_tab[et])
            agg[et] = _sc_agg_call(Ep, NP[s], NP[d], H,
                                   src, dst, ex, den, xp_chunks[s])

        wk, bk = hp['k_lin']
        q = hp['q']
        if li < 2:
            g, bb = params['norm1' if li == 0 else 'norm2']
        else:
            g = bb = jnp.zeros((128,), F32)
        new_h = {}
        for t in cur:
            K_ets = dst_roles[t]
            if not K_ets:
                continue
            octs = [agg[et] for et in K_ets]
            st = _sem_call(octs, wk.T, bk.reshape(1, 128), N[t], NP[t])
            new_h[t] = _combine_call(octs, st, q.reshape(1, 128),
                                     g.reshape(1, 128), bb.reshape(1, 128),
                                     N[t], NP[t], do_norm=(li < 2))
        h = new_h

    os_ = []
    for t in ('inst', 'var', 'array'):
        p = params['st_' + t]
        in_w, in_b = p['in_w'], p['in_b']
        ow, ob = p['out']
        os_.append(_st_call(
            h[t], N[t], NP[t],
            in_w[:128].T, in_b[:128].reshape(1, 128),
            in_w[128:256].T, in_b[128:256].reshape(1, 128),
            in_w[256:].T, in_b[256:].reshape(1, 128),
            p['inducing'], ow.T, ob.reshape(1, 128)))
    fcs = [(params['st_' + t]['fc'][0].T,
            params['st_' + t]['fc'][1].reshape(1, 128))
           for t in ('inst', 'var', 'array')]
    cws = [(params['fc_' + t][0].T, params['fc_' + t][1].reshape(1, 128))
           for t in ('inst', 'var', 'array')]
    wg, bg = params['fc_graph']
    out = _head_call(os_, fcs, cws, wg.T, bg.reshape(1, 128))
    return out.reshape(128)
